# baseline (speedup 1.0000x reference)
import functools

import jax
import jax.numpy as jnp
from jax import lax
from jax.experimental import pallas as pl
from jax.experimental.pallas import tpu as pltpu
from jax.experimental.pallas import tpu_sc as plsc

N = 10000
E = 320000
D = 128
DOUT = 64

NC = 2
NS = 16
NW = NC * NS
CH = 64
NCHUNK = 160
EPT = NCHUNK * CH
E_PAD = NW * EPT
N_PAD = 10240
RPT = N_PAD // NS
DW = 16

_MESH = plsc.VectorSubcoreMesh(core_axis_name="c", subcore_axis_name="s")


@functools.partial(
    pl.kernel,
    out_type=jax.ShapeDtypeStruct((NC, N_PAD, DW), jnp.float32),
    mesh=_MESH,
    scratch_types=[
        pltpu.VMEM((NCHUNK // 2, CH), jnp.int32),
        pltpu.VMEM((CH, DW), jnp.float32),
        pltpu.VMEM_SHARED((N_PAD, DW), jnp.float32),
    ],
    compiler_params=pltpu.CompilerParams(use_tc_tiling_on_sc=False),
)
def _sc_deg(dst_hbm, zeros_hbm, degp_hbm, didx, ones_v, deg_sh):
    cid = lax.axis_index("c")
    sid = lax.axis_index("s")
    wid = sid * NC + cid

    one = jnp.full((16,), 1.0, jnp.float32)

    def fill(r, carry):
        ones_v[r, :] = one
        return carry

    lax.fori_loop(0, CH, fill, 0)
    pltpu.sync_copy(zeros_hbm, deg_sh.at[pl.ds(sid * RPT, RPT)])
    plsc.subcore_barrier()

    def body(j, carry):
        pltpu.sync_copy(ones_v, deg_sh.at[didx.at[j]], add=True)
        return carry

    for h in range(2):
        pltpu.sync_copy(dst_hbm.at[wid, h], didx)
        lax.fori_loop(0, NCHUNK // 2, body, 0)
    plsc.subcore_barrier()
    pltpu.sync_copy(deg_sh.at[pl.ds(sid * RPT, RPT)],
                    degp_hbm.at[cid, pl.ds(sid * RPT, RPT)])


NH = NCHUNK // 2
NB = 4


@functools.partial(
    pl.kernel,
    out_type=jax.ShapeDtypeStruct((NC, N_PAD, D), jnp.float32),
    mesh=_MESH,
    scratch_types=[
        pltpu.VMEM((NH, CH), jnp.int32),
        pltpu.VMEM((NH, CH), jnp.int32),
        pltpu.VMEM((NB, CH, D), jnp.float32),
        pltpu.VMEM_SHARED((N_PAD, D), jnp.float32),
        pltpu.SemaphoreType.DMA,
        pltpu.SemaphoreType.DMA,
        pltpu.SemaphoreType.DMA,
        pltpu.SemaphoreType.DMA,
    ],
    compiler_params=pltpu.CompilerParams(
        use_tc_tiling_on_sc=False,
        internal_scratch_in_bytes=64 * 1024,
    ),
)
def _sc_agg(g_hbm, src_hbm, dst_hbm, zeros_hbm, aggp_hbm,
            sidx, didx, rows, agg_sh, sg0, sg1, ss0, ss1):
    cid = lax.axis_index("c")
    sid = lax.axis_index("s")
    wid = sid * NC + cid
    sgs = (sg0, sg1)
    sss = (ss0, ss1)

    pltpu.sync_copy(zeros_hbm, agg_sh.at[pl.ds(sid * RPT, RPT)])
    plsc.subcore_barrier()

    def fire_g(c, b, p):
        pltpu.async_copy(g_hbm.at[sidx.at[c]], rows.at[b], sgs[p])

    def drain_g(b, p):
        pltpu.make_async_copy(g_hbm.at[sidx.at[0]], rows.at[b],
                              sgs[p]).wait()

    def fire_s(c, b, p):
        pltpu.async_copy(rows.at[b], agg_sh.at[didx.at[c]], sss[p],
                         add=True)

    def drain_s(b, p):
        pltpu.make_async_copy(rows.at[b], agg_sh.at[didx.at[0]],
                              sss[p]).wait()

    def step(j, k, drain_sc, fire_next):
        b = k % NB
        p = k % 2
        drain_g(b, p)
        if drain_sc:
            drain_s((k - 2) % NB, p)
        fire_s(j, b, p)
        if fire_next:
            fire_g(j + 2, (k + 2) % NB, p)

    for h in range(2):
        pltpu.sync_copy(src_hbm.at[wid, h], sidx)
        pltpu.sync_copy(dst_hbm.at[wid, h], didx)
        fire_g(0, 0, 0)
        fire_g(1, 1, 1)
        step(0, 0, False, True)
        step(1, 1, False, True)

        def body(u, carry):
            for k in range(NB):
                step(4 * u + 2 + k, 2 + k, True, True)
            return carry

        lax.fori_loop(0, (NH - 4) // NB, body, 0)
        step(NH - 2, NH - 2, True, False)
        step(NH - 1, NH - 1, True, False)
        drain_s((NH - 2) % NB, (NH - 2) % 2)
        drain_s((NH - 1) % NB, (NH - 1) % 2)

    plsc.subcore_barrier()
    pltpu.sync_copy(agg_sh.at[pl.ds(sid * RPT, RPT)],
                    aggp_hbm.at[cid, pl.ds(sid * RPT, RPT)])


_R = 1000


def _dinv_from(deg_ref):
    deg = deg_ref[0, :, 0:1] + deg_ref[1, :, 0:1]
    return lax.rsqrt(jnp.maximum(deg, 1.0))


def _tc_h0_body(f_ref, w_ref, h0_ref):
    h0_ref[...] = jnp.maximum(
        jnp.dot(f_ref[...], w_ref[...], preferred_element_type=jnp.float32),
        0.0)


def _tc_scale_body(deg_ref, h0_ref, g_ref):
    g_ref[...] = h0_ref[...] * _dinv_from(deg_ref)


def _tc_mid_body(aggp_ref, deg_ref, h0_ref, w_ref, g2_ref):
    dinv = _dinv_from(deg_ref)
    x = (aggp_ref[0] + aggp_ref[1]) * dinv
    h1 = jnp.maximum(
        jnp.dot(x, w_ref[...], preferred_element_type=jnp.float32)
        + h0_ref[...], 0.0)
    g2_ref[...] = h1 * dinv


def _tc_out_body(aggp_ref, deg_ref, h0_ref, w_ref, wo_ref, out_ref):
    dinv = _dinv_from(deg_ref)
    x = (aggp_ref[0] + aggp_ref[1]) * dinv
    h2 = jnp.maximum(
        jnp.dot(x, w_ref[...], preferred_element_type=jnp.float32)
        + h0_ref[...], 0.0)
    out_ref[...] = jnp.dot(h2, wo_ref[...], preferred_element_type=jnp.float32)


_AGG_SPEC = pl.BlockSpec((NC, _R, D), lambda i: (0, i, 0))
_DEG_SPEC = pl.BlockSpec((NC, _R, DW), lambda i: (0, i, 0))
_ROW_SPEC = pl.BlockSpec((_R, D), lambda i: (i, 0))
_W_SPEC = pl.BlockSpec((D, D), lambda i: (0, 0))
_ROW_SDS = jax.ShapeDtypeStruct((N, D), jnp.float32)


def _tc_h0(features, W_in):
    return pl.pallas_call(
        _tc_h0_body,
        grid=(N // _R,),
        in_specs=[_ROW_SPEC, _W_SPEC],
        out_specs=_ROW_SPEC,
        out_shape=_ROW_SDS,
    )(features, W_in)


def _tc_scale(degp, h0):
    return pl.pallas_call(
        _tc_scale_body,
        grid=(N // _R,),
        in_specs=[_DEG_SPEC, _ROW_SPEC],
        out_specs=_ROW_SPEC,
        out_shape=_ROW_SDS,
    )(degp, h0)


def _tc_mid(aggp, degp, h0, W):
    return pl.pallas_call(
        _tc_mid_body,
        grid=(N // _R,),
        in_specs=[_AGG_SPEC, _DEG_SPEC, _ROW_SPEC, _W_SPEC],
        out_specs=_ROW_SPEC,
        out_shape=_ROW_SDS,
    )(aggp, degp, h0, W)


def _tc_out(aggp, degp, h0, W, W_out):
    return pl.pallas_call(
        _tc_out_body,
        grid=(N // _R,),
        in_specs=[_AGG_SPEC, _DEG_SPEC, _ROW_SPEC, _W_SPEC,
                  pl.BlockSpec((D, DOUT), lambda i: (0, 0))],
        out_specs=pl.BlockSpec((_R, DOUT), lambda i: (i, 0)),
        out_shape=jax.ShapeDtypeStruct((N, DOUT), jnp.float32),
    )(aggp, degp, h0, W, W_out)


def kernel(features, edge_index, W_in, Ws, W_out):
    src = edge_index[0]
    dst = edge_index[1]
    npad = E_PAD - E
    pad_src = jnp.arange(npad, dtype=jnp.int32) % N
    pad_dst = N + jnp.arange(npad, dtype=jnp.int32) % (N_PAD - N)
    srcp = jnp.concatenate([src, pad_src]).reshape(NW, 2, NCHUNK // 2, CH)
    dstp = jnp.concatenate([dst, pad_dst]).reshape(NW, 2, NCHUNK // 2, CH)
    zeros_d = jnp.zeros((RPT, D), jnp.float32)
    zeros_w = jnp.zeros((RPT, DW), jnp.float32)

    degp = _sc_deg(dstp, zeros_w)
    h0 = _tc_h0(features, W_in)
    g1 = _tc_scale(degp, h0)
    aggp1 = _sc_agg(g1, srcp, dstp, zeros_d)
    g2 = _tc_mid(aggp1, degp, h0, Ws[0])
    aggp2 = _sc_agg(g2, srcp, dstp, zeros_d)
    return _tc_out(aggp2, degp, h0, Ws[1], W_out)


# speedup vs baseline: 1.0457x; 1.0457x over previous
import functools

import jax
import jax.numpy as jnp
from jax import lax
from jax.experimental import pallas as pl
from jax.experimental.pallas import tpu as pltpu
from jax.experimental.pallas import tpu_sc as plsc

N = 10000
E = 320000
D = 128
DOUT = 64

NC = 2
NS = 16
NW = NC * NS
CH = 64
NCHUNK = 160
EPT = NCHUNK * CH
E_PAD = NW * EPT
N_PAD = 10240
RPT = N_PAD // NS
DW = 16

VROWS = E // 128
VPT = EPT // 128
VRT = VPT // 2

_MESH = plsc.VectorSubcoreMesh(core_axis_name="c", subcore_axis_name="s")


@functools.partial(
    pl.kernel,
    out_type=jax.ShapeDtypeStruct((NC, N_PAD, DW), jnp.float32),
    mesh=_MESH,
    scratch_types=[
        pltpu.VMEM((VRT, 2, 128), jnp.int32),
        pltpu.VMEM((CH, DW), jnp.float32),
        pltpu.VMEM_SHARED((N_PAD, DW), jnp.float32),
    ],
    compiler_params=pltpu.CompilerParams(use_tc_tiling_on_sc=False),
)
def _sc_deg(ei_hbm, zeros_hbm, degp_hbm, eidx, ones_v, deg_sh):
    cid = lax.axis_index("c")
    sid = lax.axis_index("s")
    wid = sid * NC + cid

    one = jnp.full((16,), 1.0, jnp.float32)
    lanes = lax.iota(jnp.int32, 16)

    def fill(r, carry):
        ones_v[r, :] = one
        return carry

    lax.fori_loop(0, CH, fill, 0)
    pltpu.sync_copy(zeros_hbm, deg_sh.at[pl.ds(sid * RPT, RPT)])
    plsc.subcore_barrier()

    def refill(h):
        vv = wid * VPT + h * VRT
        vc = jnp.minimum(vv, VROWS - VRT)
        pltpu.sync_copy(ei_hbm.at[pl.ds(vc, VRT)], eidx)
        n_pad_rows = jnp.minimum(vv - vc, VRT)

        def padfill(r, carry):
            for k in range(8):
                eidx[r, 1, pl.ds(16 * k, 16)] = N + (
                    (r * 128 + 16 * k + lanes) & 127)
            return carry

        lax.fori_loop(0, n_pad_rows, padfill, 0)

    def body(j, carry):
        pltpu.sync_copy(ones_v, deg_sh.at[eidx.at[j // 2, 1,
                                                  pl.ds(0, CH)]], add=True)
        pltpu.sync_copy(ones_v, deg_sh.at[eidx.at[j // 2, 1,
                                                  pl.ds(CH, CH)]], add=True)
        return carry

    for h in range(2):
        refill(h)
        lax.fori_loop(0, VRT, lambda r, c: body(2 * r, c), 0)
    plsc.subcore_barrier()
    pltpu.sync_copy(deg_sh.at[pl.ds(sid * RPT, RPT)],
                    degp_hbm.at[cid, pl.ds(sid * RPT, RPT)])


NH = NCHUNK // 2
NB = 4


@functools.partial(
    pl.kernel,
    out_type=jax.ShapeDtypeStruct((NC, N_PAD, D), jnp.float32),
    mesh=_MESH,
    scratch_types=[
        pltpu.VMEM((VRT, 2, 128), jnp.int32),
        pltpu.VMEM((NB, CH, D), jnp.float32),
        pltpu.VMEM_SHARED((N_PAD, D), jnp.float32),
        pltpu.SemaphoreType.DMA,
        pltpu.SemaphoreType.DMA,
        pltpu.SemaphoreType.DMA,
        pltpu.SemaphoreType.DMA,
    ],
    compiler_params=pltpu.CompilerParams(
        use_tc_tiling_on_sc=False,
        internal_scratch_in_bytes=64 * 1024,
    ),
)
def _sc_agg(g_hbm, ei_hbm, zeros_hbm, aggp_hbm,
            eidx, rows, agg_sh, sg0, sg1, ss0, ss1):
    cid = lax.axis_index("c")
    sid = lax.axis_index("s")
    wid = sid * NC + cid
    sgs = (sg0, sg1)
    sss = (ss0, ss1)
    lanes = lax.iota(jnp.int32, 16)

    pltpu.sync_copy(zeros_hbm, agg_sh.at[pl.ds(sid * RPT, RPT)])
    plsc.subcore_barrier()

    def sref(j, k):
        return eidx.at[j // 2, 0, pl.ds(CH * (k % 2), CH)]

    def dref(j, k):
        return eidx.at[j // 2, 1, pl.ds(CH * (k % 2), CH)]

    def fire_g(j, k, b, p):
        pltpu.async_copy(g_hbm.at[sref(j, k)], rows.at[b], sgs[p])

    def drain_g(b, p):
        pltpu.make_async_copy(g_hbm.at[eidx.at[0, 0, pl.ds(0, CH)]],
                              rows.at[b], sgs[p]).wait()

    def fire_s(j, k, b, p):
        pltpu.async_copy(rows.at[b], agg_sh.at[dref(j, k)], sss[p],
                         add=True)

    def drain_s(b, p):
        pltpu.make_async_copy(rows.at[b], agg_sh.at[eidx.at[0, 1,
                                                            pl.ds(0, CH)]],
                              sss[p]).wait()

    def step(j, k, drain_sc, fire_next):
        b = k % NB
        p = k % 2
        drain_g(b, p)
        if drain_sc:
            drain_s((k - 2) % NB, p)
        fire_s(j, k, b, p)
        if fire_next:
            fire_g(j + 2, k + 2, (k + 2) % NB, p)

    def refill(h):
        vv = wid * VPT + h * VRT
        vc = jnp.minimum(vv, VROWS - VRT)
        pltpu.sync_copy(ei_hbm.at[pl.ds(vc, VRT)], eidx)
        n_pad_rows = jnp.minimum(vv - vc, VRT)

        def padfill(r, carry):
            for k in range(8):
                v = r * 128 + 16 * k + lanes
                eidx[r, 0, pl.ds(16 * k, 16)] = v & 8191
                eidx[r, 1, pl.ds(16 * k, 16)] = N + (v & 127)
            return carry

        lax.fori_loop(0, n_pad_rows, padfill, 0)

    for h in range(2):
        refill(h)
        fire_g(0, 0, 0, 0)
        fire_g(1, 1, 1, 1)
        step(0, 0, False, True)
        step(1, 1, False, True)

        def body(u, carry):
            for k in range(NB):
                step(4 * u + 2 + k, 2 + k, True, True)
            return carry

        lax.fori_loop(0, (NH - 4) // NB, body, 0)
        step(NH - 2, NH - 2, True, False)
        step(NH - 1, NH - 1, True, False)
        drain_s((NH - 2) % NB, (NH - 2) % 2)
        drain_s((NH - 1) % NB, (NH - 1) % 2)

    plsc.subcore_barrier()
    pltpu.sync_copy(agg_sh.at[pl.ds(sid * RPT, RPT)],
                    aggp_hbm.at[cid, pl.ds(sid * RPT, RPT)])


_R = 1000


def _dinv_from(deg_ref):
    deg = deg_ref[0, :, 0:1] + deg_ref[1, :, 0:1]
    return lax.rsqrt(jnp.maximum(deg, 1.0))


def _tc_h0_body(f_ref, w_ref, h0_ref):
    h0_ref[...] = jnp.maximum(
        jnp.dot(f_ref[...], w_ref[...], preferred_element_type=jnp.float32),
        0.0)


def _tc_scale_body(deg_ref, h0_ref, g_ref):
    g_ref[...] = h0_ref[...] * _dinv_from(deg_ref)


def _tc_mid_body(aggp_ref, deg_ref, h0_ref, w_ref, g2_ref):
    dinv = _dinv_from(deg_ref)
    x = (aggp_ref[0] + aggp_ref[1]) * dinv
    h1 = jnp.maximum(
        jnp.dot(x, w_ref[...], preferred_element_type=jnp.float32)
        + h0_ref[...], 0.0)
    g2_ref[...] = h1 * dinv


def _tc_out_body(aggp_ref, deg_ref, h0_ref, w_ref, wo_ref, out_ref):
    dinv = _dinv_from(deg_ref)
    x = (aggp_ref[0] + aggp_ref[1]) * dinv
    h2 = jnp.maximum(
        jnp.dot(x, w_ref[...], preferred_element_type=jnp.float32)
        + h0_ref[...], 0.0)
    out_ref[...] = jnp.dot(h2, wo_ref[...], preferred_element_type=jnp.float32)


_AGG_SPEC = pl.BlockSpec((NC, _R, D), lambda i: (0, i, 0))
_DEG_SPEC = pl.BlockSpec((NC, _R, DW), lambda i: (0, i, 0))
_ROW_SPEC = pl.BlockSpec((_R, D), lambda i: (i, 0))
_W_SPEC = pl.BlockSpec((D, D), lambda i: (0, 0))
_ROW_SDS = jax.ShapeDtypeStruct((N, D), jnp.float32)


def _tc_h0(features, W_in):
    return pl.pallas_call(
        _tc_h0_body,
        grid=(N // _R,),
        in_specs=[_ROW_SPEC, _W_SPEC],
        out_specs=_ROW_SPEC,
        out_shape=_ROW_SDS,
    )(features, W_in)


def _tc_scale(degp, h0):
    return pl.pallas_call(
        _tc_scale_body,
        grid=(N // _R,),
        in_specs=[_DEG_SPEC, _ROW_SPEC],
        out_specs=_ROW_SPEC,
        out_shape=_ROW_SDS,
    )(degp, h0)


def _tc_mid(aggp, degp, h0, W):
    return pl.pallas_call(
        _tc_mid_body,
        grid=(N // _R,),
        in_specs=[_AGG_SPEC, _DEG_SPEC, _ROW_SPEC, _W_SPEC],
        out_specs=_ROW_SPEC,
        out_shape=_ROW_SDS,
    )(aggp, degp, h0, W)


def _tc_out(aggp, degp, h0, W, W_out):
    return pl.pallas_call(
        _tc_out_body,
        grid=(N // _R,),
        in_specs=[_AGG_SPEC, _DEG_SPEC, _ROW_SPEC, _W_SPEC,
                  pl.BlockSpec((D, DOUT), lambda i: (0, 0))],
        out_specs=pl.BlockSpec((_R, DOUT), lambda i: (i, 0)),
        out_shape=jax.ShapeDtypeStruct((N, DOUT), jnp.float32),
    )(aggp, degp, h0, W, W_out)


def kernel(features, edge_index, W_in, Ws, W_out):
    ei_v = jnp.transpose(edge_index.reshape(2, VROWS, 128), (1, 0, 2))
    zeros_d = jnp.zeros((RPT, D), jnp.float32)
    zeros_w = jnp.zeros((RPT, DW), jnp.float32)

    degp = _sc_deg(ei_v, zeros_w)
    h0 = _tc_h0(features, W_in)
    g1 = _tc_scale(degp, h0)
    aggp1 = _sc_agg(g1, ei_v, zeros_d)
    g2 = _tc_mid(aggp1, degp, h0, Ws[0])
    aggp2 = _sc_agg(g2, ei_v, zeros_d)
    return _tc_out(aggp2, degp, h0, Ws[1], W_out)
